# baseline (speedup 1.0000x reference)
import functools

import jax
import jax.numpy as jnp
from jax import lax
from jax.experimental import pallas as pl
from jax.experimental.pallas import tpu as pltpu
from jax.experimental.pallas import tpu_sc as plsc


def _sc_info():
    try:
        info = plsc.get_sparse_core_info()
        return info.num_cores, info.num_subcores
    except Exception:
        return 2, 16


@functools.partial(jax.jit, static_argnames=("nw", "steps", "seq", "d"))
def _gather_pool(idx3, table, *, nw, steps, seq, d):
    ipg = 2 * seq
    pairs_total = nw * steps
    mesh = plsc.VectorSubcoreMesh(core_axis_name="c", subcore_axis_name="s")

    nbuf = 4
    nflush = 2
    steps_per_flush = steps // nflush
    assert steps_per_flush % nbuf == 0

    @functools.partial(
        pl.kernel,
        out_type=(
            jax.ShapeDtypeStruct((pairs_total, d), jnp.float32),
            jax.ShapeDtypeStruct((pairs_total, d), jnp.float32),
        ),
        mesh=mesh,
        compiler_params=pltpu.CompilerParams(use_tc_tiling_on_sc=False),
        scratch_types=[
            pltpu.VMEM((steps, ipg), jnp.int32),
            pltpu.VMEM((nbuf, ipg, d), jnp.float32),
            pltpu.VMEM((steps_per_flush, d), jnp.float32),
            pltpu.VMEM((steps_per_flush, d), jnp.float32),
            pltpu.SemaphoreType.DMA,
        ],
    )
    def k(idx_hbm, table_hbm, out1_hbm, out2_hbm, idx_v, rows_v, s1_v, s2_v, sem):
        c = lax.axis_index("c")
        s = lax.axis_index("s")
        w = s * 2 + c

        pltpu.sync_copy(idx_hbm.at[w], idx_v)
        for b in range(nbuf):
            pltpu.async_copy(table_hbm.at[idx_v.at[b]], rows_v.at[b], sem)

        for h in range(nflush):

            def body(g, carry, h=h):
                for b in range(nbuf):
                    jloc = nbuf * g + b
                    j = h * steps_per_flush + jloc
                    pltpu.make_async_copy(
                        table_hbm.at[idx_v.at[j]], rows_v.at[b], sem
                    ).wait()
                    for snt, slab in ((0, s1_v), (1, s2_v)):
                        srow = seq * snt
                        for kk in range(d // 16):
                            col = pl.ds(16 * kk, 16)
                            acc = rows_v[b, srow, col]
                            for r in range(1, seq):
                                acc = acc + rows_v[b, srow + r, col]
                            slab[jloc, col] = acc

                    @pl.when(j + nbuf < steps)
                    def _():
                        pltpu.async_copy(
                            table_hbm.at[idx_v.at[j + nbuf]], rows_v.at[b], sem
                        )
                return carry

            lax.fori_loop(0, steps_per_flush // nbuf, body, 0)
            out_off = w * steps + h * steps_per_flush
            pltpu.sync_copy(s1_v, out1_hbm.at[pl.ds(out_off, steps_per_flush)])
            pltpu.sync_copy(s2_v, out2_hbm.at[pl.ds(out_off, steps_per_flush)])

    return k(idx3, table)


@functools.partial(jax.jit, static_argnames=("seq",))
def _project_cosine(pooled1, pooled2, Wt, b2, *, seq):
    B, d = pooled1.shape
    p = Wt.shape[1]
    blk = 1024
    inv = 1.0 / float(seq)

    def body(s1_ref, s2_ref, wt_ref, b_ref, out_ref):
        wt = wt_ref[...]
        bb = b_ref[...]
        s1 = s1_ref[...] * inv
        s2 = s2_ref[...] * inv
        p1 = jnp.dot(s1, wt, preferred_element_type=jnp.float32) + bb
        p2 = jnp.dot(s2, wt, preferred_element_type=jnp.float32) + bb
        d12 = jnp.sum(p1 * p2, axis=1)
        n1 = jnp.maximum(jnp.sqrt(jnp.sum(p1 * p1, axis=1)), 1e-8)
        n2 = jnp.maximum(jnp.sqrt(jnp.sum(p2 * p2, axis=1)), 1e-8)
        out_ref[...] = (d12 / (n1 * n2)).reshape(blk, 1)

    out = pl.pallas_call(
        body,
        grid=(B // blk,),
        in_specs=[
            pl.BlockSpec((blk, d), lambda i: (i, 0)),
            pl.BlockSpec((blk, d), lambda i: (i, 0)),
            pl.BlockSpec((d, p), lambda i: (0, 0)),
            pl.BlockSpec((1, p), lambda i: (0, 0)),
        ],
        out_specs=pl.BlockSpec((blk, 1), lambda i: (i, 0)),
        out_shape=jax.ShapeDtypeStruct((B, 1), jnp.float32),
    )(pooled1, pooled2, Wt, b2)
    return out.reshape(B)


def kernel(x, table, W, b):
    B, two, seq = x.shape
    assert two == 2
    d = table.shape[1]
    nc, ns = _sc_info()
    nw = nc * ns

    steps = B // nw
    assert steps * nw == B

    idx3 = x.astype(jnp.int32).reshape(nw, steps, 2 * seq)

    pooled1, pooled2 = _gather_pool(idx3, table, nw=nw, steps=steps, seq=seq, d=d)

    Wt = W.T
    b2 = b.reshape(1, -1)
    return _project_cosine(pooled1, pooled2, Wt, b2, seq=seq)


# speedup vs baseline: 1.3070x; 1.3070x over previous
import functools

import jax
import jax.numpy as jnp
from jax import lax
from jax.experimental import pallas as pl
from jax.experimental.pallas import tpu as pltpu
from jax.experimental.pallas import tpu_sc as plsc


def _sc_info():
    try:
        info = plsc.get_sparse_core_info()
        return info.num_cores, info.num_subcores
    except Exception:
        return 2, 16


@functools.partial(jax.jit, static_argnames=("nw", "steps", "seq", "d"))
def _gather_pool(idx3, table, *, nw, steps, seq, d):
    ipg = 2 * seq
    pairs_total = nw * steps
    mesh = plsc.VectorSubcoreMesh(core_axis_name="c", subcore_axis_name="s")

    nbuf = 4
    nflush = 2
    steps_per_flush = steps // nflush
    assert steps_per_flush % nbuf == 0

    @functools.partial(
        pl.kernel,
        out_type=(
            jax.ShapeDtypeStruct((pairs_total, d), jnp.float32),
            jax.ShapeDtypeStruct((pairs_total, d), jnp.float32),
        ),
        mesh=mesh,
        compiler_params=pltpu.CompilerParams(use_tc_tiling_on_sc=False),
        scratch_types=[
            pltpu.VMEM((steps, ipg), jnp.int32),
            pltpu.VMEM((nbuf, ipg, d), jnp.float32),
            pltpu.VMEM((steps_per_flush, d), jnp.float32),
            pltpu.VMEM((steps_per_flush, d), jnp.float32),
            pltpu.SemaphoreType.DMA,
        ],
    )
    def k(idx_hbm, table_hbm, out1_hbm, out2_hbm, idx_v, rows_v, s1_v, s2_v, sem):
        c = lax.axis_index("c")
        s = lax.axis_index("s")
        w = s * 2 + c

        pltpu.sync_copy(idx_hbm.at[w], idx_v)
        for b in range(nbuf):
            pltpu.async_copy(table_hbm.at[idx_v.at[b]], rows_v.at[b], sem)

        for h in range(nflush):

            def body(g, carry, h=h):
                for b in range(nbuf):
                    jloc = nbuf * g + b
                    j = h * steps_per_flush + jloc
                    pltpu.make_async_copy(
                        table_hbm.at[idx_v.at[j]], rows_v.at[b], sem
                    ).wait()
                    for snt, slab in ((0, s1_v), (1, s2_v)):
                        srow = seq * snt
                        for kk in range(d // 16):
                            col = pl.ds(16 * kk, 16)
                            accs = [rows_v[b, srow + i, col] for i in range(4)]
                            for base in range(4, seq, 4):
                                for i in range(4):
                                    if base + i < seq:
                                        accs[i] = accs[i] + rows_v[b, srow + base + i, col]
                            slab[jloc, col] = (accs[0] + accs[1]) + (accs[2] + accs[3])

                    @pl.when(j + nbuf < steps)
                    def _():
                        pltpu.async_copy(
                            table_hbm.at[idx_v.at[j + nbuf]], rows_v.at[b], sem
                        )
                return carry

            lax.fori_loop(0, steps_per_flush // nbuf, body, 0)
            out_off = w * steps + h * steps_per_flush
            pltpu.sync_copy(s1_v, out1_hbm.at[pl.ds(out_off, steps_per_flush)])
            pltpu.sync_copy(s2_v, out2_hbm.at[pl.ds(out_off, steps_per_flush)])

    return k(idx3, table)


@functools.partial(jax.jit, static_argnames=("seq",))
def _project_cosine(pooled1, pooled2, Wt, b2, *, seq):
    B, d = pooled1.shape
    p = Wt.shape[1]
    blk = 1024
    inv = 1.0 / float(seq)

    def body(s1_ref, s2_ref, wt_ref, b_ref, out_ref):
        wt = wt_ref[...]
        bb = b_ref[...]
        s1 = s1_ref[...] * inv
        s2 = s2_ref[...] * inv
        p1 = jnp.dot(s1, wt, preferred_element_type=jnp.float32) + bb
        p2 = jnp.dot(s2, wt, preferred_element_type=jnp.float32) + bb
        d12 = jnp.sum(p1 * p2, axis=1)
        n1 = jnp.maximum(jnp.sqrt(jnp.sum(p1 * p1, axis=1)), 1e-8)
        n2 = jnp.maximum(jnp.sqrt(jnp.sum(p2 * p2, axis=1)), 1e-8)
        out_ref[...] = (d12 / (n1 * n2)).reshape(blk, 1)

    out = pl.pallas_call(
        body,
        grid=(B // blk,),
        in_specs=[
            pl.BlockSpec((blk, d), lambda i: (i, 0)),
            pl.BlockSpec((blk, d), lambda i: (i, 0)),
            pl.BlockSpec((d, p), lambda i: (0, 0)),
            pl.BlockSpec((1, p), lambda i: (0, 0)),
        ],
        out_specs=pl.BlockSpec((blk, 1), lambda i: (i, 0)),
        out_shape=jax.ShapeDtypeStruct((B, 1), jnp.float32),
    )(pooled1, pooled2, Wt, b2)
    return out.reshape(B)


def kernel(x, table, W, b):
    B, two, seq = x.shape
    assert two == 2
    d = table.shape[1]
    nc, ns = _sc_info()
    nw = nc * ns

    steps = B // nw
    assert steps * nw == B

    idx3 = x.astype(jnp.int32).reshape(nw, steps, 2 * seq)

    pooled1, pooled2 = _gather_pool(idx3, table, nw=nw, steps=steps, seq=seq, d=d)

    Wt = W.T
    b2 = b.reshape(1, -1)
    return _project_cosine(pooled1, pooled2, Wt, b2, seq=seq)
